# baseline (speedup 1.0000x reference)
import functools

import jax
import jax.numpy as jnp
from jax import lax
from jax.experimental import pallas as pl
from jax.experimental.pallas import tpu as pltpu
from jax.experimental.pallas import tpu_sc as plsc

NC = 2
NS = 16
NW = NC * NS
DIM = 32
BATCH = 16384
VOCAB = 1000000
B_PER_W = BATCH // NW
CHUNK = 128
NCHUNK = B_PER_W // CHUNK
BLK = 40960
GRID = (VOCAB + BLK - 1) // BLK

P1CH = 1024
P1K = 21
PSPLIT = NW * P1K * P1CH
BLK_B = 32768
GRID_B = (VOCAB - PSPLIT + BLK_B - 1) // BLK_B
LEN_B = VOCAB - PSPLIT


def _p1e_body(ent_ref, we_ref, ys_ref, yo_ref):
    eo = jnp.dot(we_ref[...], ent_ref[...], preferred_element_type=jnp.float32)
    ys_ref[...] = eo[0]
    yo_ref[...] = eo[1]


def _p1b_body(pred_ref, wp_ref, ypb_ref):
    ypb_ref[...] = jnp.dot(wp_ref[...], pred_ref[...],
                           preferred_element_type=jnp.float32)[0]


def _p1p_body(pred3_hbm, wp_hbm, yp_hbm, buf, acc, wv, sem0, sem1):
    wid = lax.axis_index("s") * NC + lax.axis_index("c")
    pltpu.sync_copy(wp_hbm, wv)
    wlo = wv[pl.ds(0, 16)]
    whi = wv[pl.ds(16, 16)]
    wsc = [wlo[j] for j in range(16)] + [whi[j] for j in range(16)]
    sems = (sem0, sem1)

    def fire(p, off):
        for t in range(4):
            pltpu.async_copy(
                pred3_hbm.at[t, :, pl.ds(off, P1CH)],
                buf.at[p, t], sems[p])

    def drain(p, off):
        for t in range(4):
            pltpu.make_async_copy(
                pred3_hbm.at[t, :, pl.ds(off, P1CH)],
                buf.at[p, t], sems[p]).wait()

    def compute(p):
        def grp(g, carry):
            base = g * 16
            a = jnp.zeros((16,), jnp.float32)
            for t in range(4):
                for dd in range(8):
                    v = buf[p, t, dd, pl.ds(base, 16)]
                    a = a + v * wsc[8 * t + dd]
            acc[pl.ds(base, 16)] = a
            return carry
        lax.fori_loop(0, P1CH // 16, grp, 0)

    def chunk_off(k):
        return (wid + 32 * k) * P1CH

    def do_step(p, off):
        drain(p, off)
        compute(p)
        pltpu.sync_copy(acc, yp_hbm.at[pl.ds(off, P1CH)])

    fire(0, chunk_off(0))

    def pair(k2, carry):
        k = 2 * k2
        fire(1, chunk_off(k + 1))
        do_step(0, chunk_off(k))
        fire(0, chunk_off(k + 2))
        do_step(1, chunk_off(k + 1))
        return carry

    lax.fori_loop(0, (P1K - 1) // 2, pair, 0)
    do_step(0, chunk_off(P1K - 1))


def _sc_body(idx_hbm, m_hbm, ys_hbm, ypa_hbm, ypb_hbm, yo_hbm, wb_hbm,
             out_hbm, sidx, paidx, pbidx, oidx, gs, gpa, gpb, go, mv,
             scores, wv, sem):
    wid = lax.axis_index("s") * NC + lax.axis_index("c")
    base = wid * B_PER_W

    pltpu.sync_copy(idx_hbm.at[0, wid], sidx)
    pltpu.sync_copy(idx_hbm.at[1, wid], paidx)
    pltpu.sync_copy(idx_hbm.at[2, wid], pbidx)
    pltpu.sync_copy(idx_hbm.at[3, wid], oidx)
    pltpu.sync_copy(m_hbm.at[wid], mv)
    pltpu.sync_copy(wb_hbm, wv)

    descs = []
    for k in range(NCHUNK):
        dst = pl.ds(k * CHUNK, CHUNK)
        descs.append(pltpu.async_copy(ys_hbm.at[sidx.at[k]], gs.at[dst], sem))
        descs.append(pltpu.async_copy(ypa_hbm.at[paidx.at[k]], gpa.at[dst], sem))
        descs.append(pltpu.async_copy(ypb_hbm.at[pbidx.at[k]], gpb.at[dst], sem))
        descs.append(pltpu.async_copy(yo_hbm.at[oidx.at[k]], go.at[dst], sem))
    for d in descs:
        d.wait()

    bias = wv[pl.ds(0, 16)][0]
    for v in range(B_PER_W // 16):
        sl = pl.ds(v * 16, 16)
        m = mv[sl]
        gp = m * gpa[sl] + (1.0 - m) * gpb[sl]
        scores[sl] = gs[sl] + gp + go[sl] + bias

    pltpu.sync_copy(scores, out_hbm.at[pl.ds(base, B_PER_W)])


@jax.jit
def _triple_score(idx4, m_r, ent_t, pred_t, pred3, we, wp2, wpv, wb):
    mesh = plsc.VectorSubcoreMesh(core_axis_name="c", subcore_axis_name="s")

    ypa = functools.partial(
        pl.kernel,
        out_type=jax.ShapeDtypeStruct((PSPLIT,), jnp.float32),
        mesh=mesh,
        scratch_types=[
            pltpu.VMEM((2, 4, 8, P1CH), jnp.float32),
            pltpu.VMEM((P1CH,), jnp.float32),
            pltpu.VMEM((DIM,), jnp.float32),
            pltpu.SemaphoreType.DMA,
            pltpu.SemaphoreType.DMA,
        ],
        compiler_params=pltpu.CompilerParams(use_tc_tiling_on_sc=True),
    )(_p1p_body)(pred3, wpv)

    ys, yo = pl.pallas_call(
        _p1e_body,
        grid=(GRID,),
        in_specs=[
            pl.BlockSpec((DIM, BLK), lambda i: (0, i)),
            pl.BlockSpec((2, DIM), lambda i: (0, 0)),
        ],
        out_specs=[
            pl.BlockSpec((BLK,), lambda i: (i,)),
            pl.BlockSpec((BLK,), lambda i: (i,)),
        ],
        out_shape=[
            jax.ShapeDtypeStruct((VOCAB,), jnp.float32),
            jax.ShapeDtypeStruct((VOCAB,), jnp.float32),
        ],
    )(ent_t, we)

    (ypb,) = pl.pallas_call(
        _p1b_body,
        grid=(GRID_B,),
        in_specs=[
            pl.BlockSpec((DIM, BLK_B), lambda i: (0, i + PSPLIT // BLK_B)),
            pl.BlockSpec((1, DIM), lambda i: (0, 0)),
        ],
        out_specs=[pl.BlockSpec((BLK_B,), lambda i: (i,))],
        out_shape=[jax.ShapeDtypeStruct((LEN_B,), jnp.float32)],
    )(pred_t, wp2)

    f = functools.partial(
        pl.kernel,
        out_type=jax.ShapeDtypeStruct((BATCH,), jnp.float32),
        mesh=mesh,
        scratch_types=[
            pltpu.VMEM((NCHUNK, CHUNK), jnp.int32),
            pltpu.VMEM((NCHUNK, CHUNK), jnp.int32),
            pltpu.VMEM((NCHUNK, CHUNK), jnp.int32),
            pltpu.VMEM((NCHUNK, CHUNK), jnp.int32),
            pltpu.VMEM((B_PER_W,), jnp.float32),
            pltpu.VMEM((B_PER_W,), jnp.float32),
            pltpu.VMEM((B_PER_W,), jnp.float32),
            pltpu.VMEM((B_PER_W,), jnp.float32),
            pltpu.VMEM((B_PER_W,), jnp.float32),
            pltpu.VMEM((B_PER_W,), jnp.float32),
            pltpu.VMEM((16,), jnp.float32),
            pltpu.SemaphoreType.DMA,
        ],
        compiler_params=pltpu.CompilerParams(
            needs_layout_passes=False, use_tc_tiling_on_sc=False),
    )(_sc_body)
    return f(idx4, m_r, ys, ypa, ypb, yo, wb)


def kernel(triple_ids, entity_emb, pred_emb, W, b):
    if triple_ids.ndim == 1:
        triple_ids = triple_ids[None, :]
    tids = triple_ids.astype(jnp.int32)
    s = tids[:, 0]
    p = tids[:, 1]
    o = tids[:, 2]
    pa = jnp.minimum(p, PSPLIT - 1)
    pb = jnp.clip(p - PSPLIT, 0, LEN_B - 1)
    m = (p < PSPLIT).astype(jnp.float32)
    idx4 = jnp.stack([s, pa, pb, o]).reshape(4, NW, NCHUNK, CHUNK)
    m_r = m.reshape(NW, B_PER_W)
    w3 = W.reshape(3, DIM)
    we = jnp.stack([w3[0], w3[2]])
    wp2 = w3[1].reshape(1, DIM)
    wpv = w3[1]
    wb = jnp.broadcast_to(b.reshape(1), (16,)).astype(jnp.float32)
    pred_t = pred_emb.T
    pred3 = pred_t.reshape(4, 8, VOCAB)
    return _triple_score(idx4, m_r, entity_emb.T, pred_t, pred3,
                         we, wp2, wpv, wb)


# speedup vs baseline: 1.5274x; 1.5274x over previous
import functools

import jax
import jax.numpy as jnp
from jax import lax
from jax.experimental import pallas as pl
from jax.experimental.pallas import tpu as pltpu
from jax.experimental.pallas import tpu_sc as plsc

NC = 2
NS = 16
NW = NC * NS
DIM = 32
BATCH = 16384
VOCAB = 1000000
B_PER_W = BATCH // NW
CHUNK = 128
NCHUNK = B_PER_W // CHUNK
BLK = 40960
GRID = (VOCAB + BLK - 1) // BLK


def _p1_body(ent_ref, pred_ref, we_ref, wp_ref, ys_ref, yo_ref, yp_ref):
    eo = jnp.dot(we_ref[...], ent_ref[...], preferred_element_type=jnp.float32)
    ys_ref[...] = eo[0]
    yo_ref[...] = eo[1]
    yp_ref[...] = jnp.dot(wp_ref[...], pred_ref[...],
                          preferred_element_type=jnp.float32)[0]


def _sc_body(ids_hbm, ys_hbm, yp_hbm, yo_hbm, wb_hbm, out_hbm,
             sidx, pidx, oidx, gs, gp, go, scores, wv, sem):
    wid = lax.axis_index("s") * NC + lax.axis_index("c")
    base = wid * B_PER_W

    pltpu.sync_copy(ids_hbm.at[0, wid], sidx)
    pltpu.sync_copy(ids_hbm.at[1, wid], pidx)
    pltpu.sync_copy(ids_hbm.at[2, wid], oidx)
    pltpu.sync_copy(wb_hbm, wv)

    descs = []
    for k in range(NCHUNK):
        dst = pl.ds(k * CHUNK, CHUNK)
        descs.append(pltpu.async_copy(ys_hbm.at[sidx.at[k]], gs.at[dst], sem))
        descs.append(pltpu.async_copy(yp_hbm.at[pidx.at[k]], gp.at[dst], sem))
        descs.append(pltpu.async_copy(yo_hbm.at[oidx.at[k]], go.at[dst], sem))
    for d in descs:
        d.wait()

    bias = wv[pl.ds(0, 16)][0]
    for v in range(B_PER_W // 16):
        sl = pl.ds(v * 16, 16)
        scores[sl] = gs[sl] + gp[sl] + go[sl] + bias

    pltpu.sync_copy(scores, out_hbm.at[pl.ds(base, B_PER_W)])


@jax.jit
def _triple_score(ids_r, ent_t, pred_t, we, wp, wb):
    ys, yo, yp = pl.pallas_call(
        _p1_body,
        grid=(GRID,),
        in_specs=[
            pl.BlockSpec((DIM, BLK), lambda i: (0, i)),
            pl.BlockSpec((DIM, BLK), lambda i: (0, i)),
            pl.BlockSpec((2, DIM), lambda i: (0, 0)),
            pl.BlockSpec((1, DIM), lambda i: (0, 0)),
        ],
        out_specs=[
            pl.BlockSpec((BLK,), lambda i: (i,)),
            pl.BlockSpec((BLK,), lambda i: (i,)),
            pl.BlockSpec((BLK,), lambda i: (i,)),
        ],
        out_shape=[
            jax.ShapeDtypeStruct((VOCAB,), jnp.float32),
            jax.ShapeDtypeStruct((VOCAB,), jnp.float32),
            jax.ShapeDtypeStruct((VOCAB,), jnp.float32),
        ],
    )(ent_t, pred_t, we, wp)

    mesh = plsc.VectorSubcoreMesh(core_axis_name="c", subcore_axis_name="s")
    f = functools.partial(
        pl.kernel,
        out_type=jax.ShapeDtypeStruct((BATCH,), jnp.float32),
        mesh=mesh,
        scratch_types=[
            pltpu.VMEM((NCHUNK, CHUNK), jnp.int32),
            pltpu.VMEM((NCHUNK, CHUNK), jnp.int32),
            pltpu.VMEM((NCHUNK, CHUNK), jnp.int32),
            pltpu.VMEM((B_PER_W,), jnp.float32),
            pltpu.VMEM((B_PER_W,), jnp.float32),
            pltpu.VMEM((B_PER_W,), jnp.float32),
            pltpu.VMEM((B_PER_W,), jnp.float32),
            pltpu.VMEM((16,), jnp.float32),
            pltpu.SemaphoreType.DMA,
        ],
        compiler_params=pltpu.CompilerParams(
            needs_layout_passes=False, use_tc_tiling_on_sc=False),
    )(_sc_body)
    return f(ids_r, ys, yp, yo, wb)


def kernel(triple_ids, entity_emb, pred_emb, W, b):
    if triple_ids.ndim == 1:
        triple_ids = triple_ids[None, :]
    ids_r = triple_ids.T.astype(jnp.int32).reshape(3, NW, NCHUNK, CHUNK)
    w3 = W.reshape(3, DIM)
    we = jnp.stack([w3[0], w3[2]])
    wp = w3[1].reshape(1, DIM)
    wb = jnp.broadcast_to(b.reshape(1), (16,)).astype(jnp.float32)
    return _triple_score(ids_r, entity_emb.T, pred_emb.T, we, wp, wb)
